# baseline (speedup 1.0000x reference)
import functools

import jax
import jax.numpy as jnp
from jax import lax
from jax.experimental import pallas as pl
from jax.experimental.pallas import tpu as pltpu
from jax.experimental.pallas import tpu_sc as plsc

N = 50000
E = 800000
G = 512
HID = 64
HH = 32
IN_DIM = 3

NC, NS = 2, 16
CH = 128

N_PAD = 51200
RPT = N_PAD // NS
HALF = N_PAD // NC
ZPAD = HALF + 1024
HPT = HALF // NS
E_PAD = 802816
ECH = E_PAD // CH
ECH_C = ECH // NC
ECH_CT = ECH_C // NS
G_PAD = 640
DW = 16
BCH = N_PAD // CH

BR = 6400
TC_GRID = N_PAD // BR
BPH = HALF // BR

_mesh = plsc.VectorSubcoreMesh(core_axis_name="c", subcore_axis_name="s")
_f32 = jnp.float32
_sc_params = pltpu.CompilerParams(use_tc_tiling_on_sc=False)
_sc_params_nl = pltpu.CompilerParams(use_tc_tiling_on_sc=False,
                                     needs_layout_passes=False)


@functools.partial(
    pl.kernel,
    out_type=jax.ShapeDtypeStruct((NC, N_PAD, DW), _f32),
    mesh=_mesh,
    scratch_types=[
        pltpu.VMEM_SHARED((N_PAD, DW), _f32),
        pltpu.VMEM((4, CH), jnp.int32),
        pltpu.VMEM((CH, DW), _f32),
    ],
    compiler_params=_sc_params,
)
def _deg_sc(dst2d, zeros_col, ones_col, out, dacc, dbuf, ones_v):
    c = lax.axis_index("c")
    s = lax.axis_index("s")
    base = s * RPT
    pltpu.sync_copy(zeros_col.at[pl.ds(base, RPT)], dacc.at[pl.ds(base, RPT)])
    pltpu.sync_copy(ones_col, ones_v)
    plsc.subcore_barrier()

    chunk0 = c * ECH_C + s * ECH_CT

    def body(b, carry):
        pltpu.sync_copy(dst2d.at[pl.ds(chunk0 + b * 4, 4)], dbuf)
        for j in range(4):
            pltpu.sync_copy(ones_v, dacc.at[dbuf.at[j]], add=True)
        return carry

    lax.fori_loop(0, ECH_CT // 4, body, 0)
    plsc.subcore_barrier()
    pltpu.sync_copy(dacc.at[pl.ds(base, RPT)], out.at[c, pl.ds(base, RPT)])


@functools.partial(
    pl.kernel,
    out_type=jax.ShapeDtypeStruct((NC, HALF, HID), _f32),
    mesh=_mesh,
    scratch_types=[
        pltpu.VMEM_SHARED((ZPAD, HID), _f32),
        pltpu.VMEM((2, CH), jnp.int32),
        pltpu.VMEM((2, CH), jnp.int32),
        pltpu.VMEM((2, CH, HID), _f32),
        pltpu.VMEM((16,), jnp.int32),
        pltpu.SemaphoreType.DMA,
    ],
    compiler_params=_sc_params_nl,
)
def _layer_sc(y, srcp, dstp, counts, z, zacc, sbuf, dbuf, rows, kbuf, sem):
    c = lax.axis_index("c")
    s = lax.axis_index("s")
    base = s * HPT

    pltpu.sync_copy(y.at[pl.ds(c * HALF + base, HPT)],
                    zacc.at[pl.ds(base, HPT)])
    pltpu.sync_copy(counts.at[c], kbuf)
    plsc.subcore_barrier()

    nblk = lax.reduce_max(kbuf[...], axes=(0,))
    chunk_base = s * 2 * nblk

    def body(b, carry):
        off = chunk_base + b * 2
        pltpu.sync_copy(srcp.at[c, pl.ds(off, 2)], sbuf)
        pltpu.sync_copy(dstp.at[c, pl.ds(off, 2)], dbuf)
        descs = [
            pltpu.async_copy(y.at[sbuf.at[j]], rows.at[j], sem)
            for j in range(2)
        ]
        for j in range(2):
            descs[j].wait()
            pltpu.sync_copy(rows.at[j], zacc.at[dbuf.at[j]], add=True)
        return carry

    lax.fori_loop(0, nblk, body, 0)
    plsc.subcore_barrier()
    pltpu.sync_copy(zacc.at[pl.ds(base, HPT)], z.at[c, pl.ds(base, HPT)])


@functools.partial(
    pl.kernel,
    out_type=jax.ShapeDtypeStruct((NC, G_PAD, HH), _f32),
    mesh=_mesh,
    scratch_types=[
        pltpu.VMEM_SHARED((G_PAD, HH), _f32),
        pltpu.VMEM((1, CH), jnp.int32),
        pltpu.VMEM((CH, HH), _f32),
    ],
    compiler_params=_sc_params,
)
def _pool_sc(ha, hb, batch2d, zeros_g, out, pacc, bbuf, rows):
    c = lax.axis_index("c")
    s = lax.axis_index("s")

    @pl.when(s == 0)
    def _():
        pltpu.sync_copy(zeros_g, pacc)

    plsc.subcore_barrier()

    def run(h_hbm):
        def body(b, carry):
            ci = s * (RPT // CH) + b
            pltpu.sync_copy(h_hbm.at[pl.ds(ci * CH, CH)], rows)
            pltpu.sync_copy(batch2d.at[pl.ds(ci, 1)], bbuf)
            pltpu.sync_copy(rows, pacc.at[bbuf.at[0]], add=True)
            return carry

        lax.fori_loop(0, RPT // CH, body, 0)

    @pl.when(c == 0)
    def _():
        run(ha)

    @pl.when(c == 1)
    def _():
        run(hb)

    plsc.subcore_barrier()

    @pl.when(s == 0)
    def _():
        pltpu.sync_copy(pacc, out.at[c])


def _tc_first_body(deg_ref, x_ref, w_ref, dinv_ref, y_ref):
    d = deg_ref[...]
    dinv = lax.rsqrt(d[0, :, :1] + d[1, :, :1] + 1.0)
    xw = lax.dot_general(x_ref[...], w_ref[...], (((1,), (0,)), ((), ())),
                         preferred_element_type=_f32)
    dinv_ref[...] = dinv
    y_ref[...] = xw * dinv


def _tc_layer_body(z_ref, dinv_ref, b_ref, w_ref, y_ref):
    dinv = dinv_ref[...]
    h = jnp.maximum(z_ref[0] * dinv + b_ref[...], 0.0)
    y_ref[...] = lax.dot_general(h, w_ref[...], (((1,), (0,)), ((), ())),
                                 preferred_element_type=_f32) * dinv


def _tc_hidden_body(z_ref, dinv_ref, b_ref, ha_ref, hb_ref):
    dinv = dinv_ref[...]
    h = jnp.maximum(z_ref[0] * dinv + b_ref[...], 0.0)
    ha_ref[...] = h[:, :HH]
    hb_ref[...] = h[:, HH:]


def _tc_out_body(p_ref, wl_ref, bl_ref, o_ref):
    p = p_ref[...]
    h = jnp.concatenate([p[0, :G], p[1, :G]], axis=1)
    o_ref[...] = lax.dot_general(h, wl_ref[...], (((1,), (0,)), ((), ())),
                                 preferred_element_type=_f32) + bl_ref[...]


def _row_spec(cols):
    return pl.BlockSpec((BR, cols), lambda i: (i, 0))


def _bank_spec(cols):
    return pl.BlockSpec((1, BR, cols), lambda i: (i // BPH, i % BPH, 0))


def _full_spec(shape):
    return pl.BlockSpec(shape, lambda i: tuple(0 for _ in shape))


_tc_first = pl.pallas_call(
    _tc_first_body,
    grid=(TC_GRID,),
    in_specs=[
        pl.BlockSpec((NC, BR, DW), lambda i: (0, i, 0)),
        _row_spec(8),
        _full_spec((8, HID)),
    ],
    out_specs=[_row_spec(1), _row_spec(HID)],
    out_shape=[
        jax.ShapeDtypeStruct((N_PAD, 1), _f32),
        jax.ShapeDtypeStruct((N_PAD, HID), _f32),
    ],
)

_tc_layer = pl.pallas_call(
    _tc_layer_body,
    grid=(TC_GRID,),
    in_specs=[
        _bank_spec(HID), _row_spec(1),
        _full_spec((1, HID)), _full_spec((HID, HID)),
    ],
    out_specs=_row_spec(HID),
    out_shape=jax.ShapeDtypeStruct((N_PAD, HID), _f32),
)

_tc_hidden = pl.pallas_call(
    _tc_hidden_body,
    grid=(TC_GRID,),
    in_specs=[_bank_spec(HID), _row_spec(1), _full_spec((1, HID))],
    out_specs=[_row_spec(HH), _row_spec(HH)],
    out_shape=[
        jax.ShapeDtypeStruct((N_PAD, HH), _f32),
        jax.ShapeDtypeStruct((N_PAD, HH), _f32),
    ],
)

_tc_out = pl.pallas_call(
    _tc_out_body,
    grid=(1,),
    in_specs=[
        _full_spec((NC, G_PAD, HH)),
        _full_spec((HID, 2)),
        _full_spec((1, 2)),
    ],
    out_specs=_full_spec((G, 2)),
    out_shape=jax.ShapeDtypeStruct((G, 2), _f32),
)


def kernel(x, edge_index, batch, W1, b1, W2, b2, W3, b3, W4, b4, W5, b5,
           Wl, bl):
    src = edge_index[0]
    dst = edge_index[1]
    ep = E_PAD - E
    idxv = jnp.arange(E_PAD, dtype=jnp.int32)
    src_p = jnp.concatenate([src, jnp.zeros((ep,), jnp.int32)])
    dst_p = jnp.concatenate(
        [dst, N + (jnp.arange(ep, dtype=jnp.int32) % (N_PAD - N))])

    flag = (dst_p >= HALF).astype(jnp.int32)
    cnt0 = E_PAD - jnp.sum(flag)
    order = jnp.argsort(flag, stable=True)
    ss = src_p[order]
    ds = dst_p[order]
    sacr = HALF + (idxv % (ZPAD - HALF))
    b0s = jnp.where(idxv < cnt0, ss, 0)
    b0d = jnp.where(idxv < cnt0, ds, sacr)
    cnt1 = E_PAD - cnt0
    ss1 = jnp.roll(ss, -cnt0)
    ds1 = jnp.roll(ds, -cnt0)
    b1s = jnp.where(idxv < cnt1, ss1, 0)
    b1d = jnp.where(idxv < cnt1, ds1 - HALF, sacr)
    srcp = jnp.stack([b0s, b1s]).reshape(NC, ECH, CH)
    dstp = jnp.stack([b0d, b1d]).reshape(NC, ECH, CH)
    k0 = (cnt0 + NS * CH * 2 - 1) // (NS * CH * 2)
    k1 = (cnt1 + NS * CH * 2 - 1) // (NS * CH * 2)
    counts = jnp.broadcast_to(
        jnp.stack([k0, k1]).astype(jnp.int32)[:, None], (NC, 16))

    dst2d = dst_p.reshape(ECH, CH)
    batch2d = jnp.concatenate(
        [batch, G + (jnp.arange(N_PAD - N, dtype=jnp.int32) % (G_PAD - G))]
    ).reshape(BCH, CH)

    x_p = jnp.zeros((N_PAD, 8), _f32).at[:N, :IN_DIM].set(x)
    w1_p = jnp.zeros((8, HID), _f32).at[:IN_DIM].set(W1)

    zeros_col = jnp.zeros((N_PAD, DW), _f32)
    ones_col = jnp.ones((CH, DW), _f32)
    zeros_g = jnp.zeros((G_PAD, HH), _f32)

    deg = _deg_sc(dst2d, zeros_col, ones_col)
    dinv, y = _tc_first(deg, x_p, w1_p)

    for W, b in ((W2, b1), (W3, b2), (W4, b3), (W5, b4)):
        z = _layer_sc(y, srcp, dstp, counts)
        y = _tc_layer(z, dinv, b.reshape(1, HID), W)

    z = _layer_sc(y, srcp, dstp, counts)
    ha, hb = _tc_hidden(z, dinv, b5.reshape(1, HID))

    pooled = _pool_sc(ha, hb, batch2d, zeros_g)
    return _tc_out(pooled, Wl, bl.reshape(1, 2))


# speedup vs baseline: 1.5032x; 1.5032x over previous
import functools

import jax
import jax.numpy as jnp
from jax import lax
from jax.experimental import pallas as pl
from jax.experimental.pallas import tpu as pltpu
from jax.experimental.pallas import tpu_sc as plsc

N = 50000
E = 800000
G = 512
HID = 64
HH = 32
IN_DIM = 3

NC, NS = 2, 16
CH = 128

N_PAD = 51200
RPT = N_PAD // NS
E_PAD = 802816
ECH = E_PAD // CH
ECH_T = ECH // NS
ECH_C = ECH // NC
ECH_CT = ECH_C // NS
G_PAD = 640
DW = 16
BCH = N_PAD // CH

BR = 6400
TC_GRID = N_PAD // BR

_mesh = plsc.VectorSubcoreMesh(core_axis_name="c", subcore_axis_name="s")
_f32 = jnp.float32
_sc_params = pltpu.CompilerParams(use_tc_tiling_on_sc=False)


@functools.partial(
    pl.kernel,
    out_type=jax.ShapeDtypeStruct((NC, N_PAD, DW), _f32),
    mesh=_mesh,
    scratch_types=[
        pltpu.VMEM_SHARED((N_PAD, DW), _f32),
        pltpu.VMEM((4, CH), jnp.int32),
        pltpu.VMEM((CH, DW), _f32),
    ],
    compiler_params=_sc_params,
)
def _deg_sc(dst2d, zeros_col, ones_col, out, dacc, dbuf, ones_v):
    c = lax.axis_index("c")
    s = lax.axis_index("s")
    base = s * RPT
    pltpu.sync_copy(zeros_col.at[pl.ds(base, RPT)], dacc.at[pl.ds(base, RPT)])
    pltpu.sync_copy(ones_col, ones_v)
    plsc.subcore_barrier()

    chunk0 = c * ECH_C + s * ECH_CT

    def body(b, carry):
        pltpu.sync_copy(dst2d.at[pl.ds(chunk0 + b * 4, 4)], dbuf)
        for j in range(4):
            pltpu.sync_copy(ones_v, dacc.at[dbuf.at[j]], add=True)
        return carry

    lax.fori_loop(0, ECH_CT // 4, body, 0)
    plsc.subcore_barrier()
    pltpu.sync_copy(dacc.at[pl.ds(base, RPT)], out.at[c, pl.ds(base, RPT)])


@functools.partial(
    pl.kernel,
    out_type=(
        jax.ShapeDtypeStruct((N_PAD, HH), _f32),
        jax.ShapeDtypeStruct((N_PAD, HH), _f32),
    ),
    mesh=_mesh,
    scratch_types=[
        pltpu.VMEM_SHARED((N_PAD, HH), _f32),
        pltpu.VMEM((4, CH), jnp.int32),
        pltpu.VMEM((4, CH), jnp.int32),
        pltpu.VMEM((4, CH, HH), _f32),
        pltpu.SemaphoreType.DMA,
    ],
    compiler_params=_sc_params,
)
def _layer_sc(ya, yb, src2d, dst2d, za, zb, zacc, sbuf, dbuf, rows, sem):
    c = lax.axis_index("c")
    s = lax.axis_index("s")
    base = s * RPT
    chunk0 = s * ECH_T

    def run(y_hbm, z_hbm):
        pltpu.sync_copy(y_hbm.at[pl.ds(base, RPT)], zacc.at[pl.ds(base, RPT)])
        plsc.subcore_barrier()

        def body(b, carry):
            off = chunk0 + b * 4
            pltpu.sync_copy(src2d.at[pl.ds(off, 4)], sbuf)
            pltpu.sync_copy(dst2d.at[pl.ds(off, 4)], dbuf)
            descs = [
                pltpu.async_copy(y_hbm.at[sbuf.at[j]], rows.at[j], sem)
                for j in range(4)
            ]
            for j in range(4):
                descs[j].wait()
                pltpu.sync_copy(rows.at[j], zacc.at[dbuf.at[j]], add=True)
            return carry

        lax.fori_loop(0, ECH_T // 4, body, 0)
        plsc.subcore_barrier()
        pltpu.sync_copy(zacc.at[pl.ds(base, RPT)], z_hbm.at[pl.ds(base, RPT)])

    @pl.when(c == 0)
    def _():
        run(ya, za)

    @pl.when(c == 1)
    def _():
        run(yb, zb)


@functools.partial(
    pl.kernel,
    out_type=jax.ShapeDtypeStruct((NC, G_PAD, HH), _f32),
    mesh=_mesh,
    scratch_types=[
        pltpu.VMEM_SHARED((G_PAD, HH), _f32),
        pltpu.VMEM((1, CH), jnp.int32),
        pltpu.VMEM((CH, HH), _f32),
    ],
    compiler_params=_sc_params,
)
def _pool_sc(ha, hb, batch2d, zeros_g, out, pacc, bbuf, rows):
    c = lax.axis_index("c")
    s = lax.axis_index("s")

    @pl.when(s == 0)
    def _():
        pltpu.sync_copy(zeros_g, pacc)

    plsc.subcore_barrier()

    def run(h_hbm):
        def body(b, carry):
            ci = s * (RPT // CH) + b
            pltpu.sync_copy(h_hbm.at[pl.ds(ci * CH, CH)], rows)
            pltpu.sync_copy(batch2d.at[pl.ds(ci, 1)], bbuf)
            pltpu.sync_copy(rows, pacc.at[bbuf.at[0]], add=True)
            return carry

        lax.fori_loop(0, RPT // CH, body, 0)

    @pl.when(c == 0)
    def _():
        run(ha)

    @pl.when(c == 1)
    def _():
        run(hb)

    plsc.subcore_barrier()

    @pl.when(s == 0)
    def _():
        pltpu.sync_copy(pacc, out.at[c])


def _tc_first_body(deg_ref, x_ref, w_ref, dinv_ref, ya_ref, yb_ref):
    d = deg_ref[...]
    dinv = lax.rsqrt(d[0, :, :1] + d[1, :, :1] + 1.0)
    xw = lax.dot_general(x_ref[...], w_ref[...], (((1,), (0,)), ((), ())),
                         preferred_element_type=_f32)
    y = xw * dinv
    dinv_ref[...] = dinv
    ya_ref[...] = y[:, :HH]
    yb_ref[...] = y[:, HH:]


def _tc_layer_body(za_ref, zb_ref, dinv_ref, b_ref, w_ref, ya_ref, yb_ref):
    dinv = dinv_ref[...]
    z = jnp.concatenate([za_ref[...], zb_ref[...]], axis=1)
    h = jnp.maximum(z * dinv + b_ref[...], 0.0)
    y = lax.dot_general(h, w_ref[...], (((1,), (0,)), ((), ())),
                        preferred_element_type=_f32) * dinv
    ya_ref[...] = y[:, :HH]
    yb_ref[...] = y[:, HH:]


def _tc_hidden_body(za_ref, zb_ref, dinv_ref, b_ref, ha_ref, hb_ref):
    dinv = dinv_ref[...]
    z = jnp.concatenate([za_ref[...], zb_ref[...]], axis=1)
    h = jnp.maximum(z * dinv + b_ref[...], 0.0)
    ha_ref[...] = h[:, :HH]
    hb_ref[...] = h[:, HH:]


def _tc_out_body(p_ref, wl_ref, bl_ref, o_ref):
    p = p_ref[...]
    h = jnp.concatenate([p[0, :G], p[1, :G]], axis=1)
    o_ref[...] = lax.dot_general(h, wl_ref[...], (((1,), (0,)), ((), ())),
                                 preferred_element_type=_f32) + bl_ref[...]


def _row_spec(cols):
    return pl.BlockSpec((BR, cols), lambda i: (i, 0))


def _full_spec(shape):
    return pl.BlockSpec(shape, lambda i: tuple(0 for _ in shape))


_tc_first = pl.pallas_call(
    _tc_first_body,
    grid=(TC_GRID,),
    in_specs=[
        pl.BlockSpec((NC, BR, DW), lambda i: (0, i, 0)),
        _row_spec(8),
        _full_spec((8, HID)),
    ],
    out_specs=[_row_spec(1), _row_spec(HH), _row_spec(HH)],
    out_shape=[
        jax.ShapeDtypeStruct((N_PAD, 1), _f32),
        jax.ShapeDtypeStruct((N_PAD, HH), _f32),
        jax.ShapeDtypeStruct((N_PAD, HH), _f32),
    ],
)

_tc_layer = pl.pallas_call(
    _tc_layer_body,
    grid=(TC_GRID,),
    in_specs=[
        _row_spec(HH), _row_spec(HH), _row_spec(1),
        _full_spec((1, HID)), _full_spec((HID, HID)),
    ],
    out_specs=[_row_spec(HH), _row_spec(HH)],
    out_shape=[
        jax.ShapeDtypeStruct((N_PAD, HH), _f32),
        jax.ShapeDtypeStruct((N_PAD, HH), _f32),
    ],
)

_tc_hidden = pl.pallas_call(
    _tc_hidden_body,
    grid=(TC_GRID,),
    in_specs=[
        _row_spec(HH), _row_spec(HH), _row_spec(1), _full_spec((1, HID)),
    ],
    out_specs=[_row_spec(HH), _row_spec(HH)],
    out_shape=[
        jax.ShapeDtypeStruct((N_PAD, HH), _f32),
        jax.ShapeDtypeStruct((N_PAD, HH), _f32),
    ],
)

_tc_out = pl.pallas_call(
    _tc_out_body,
    grid=(1,),
    in_specs=[
        _full_spec((NC, G_PAD, HH)),
        _full_spec((HID, 2)),
        _full_spec((1, 2)),
    ],
    out_specs=_full_spec((G, 2)),
    out_shape=jax.ShapeDtypeStruct((G, 2), _f32),
)


def kernel(x, edge_index, batch, W1, b1, W2, b2, W3, b3, W4, b4, W5, b5,
           Wl, bl):
    src = edge_index[0]
    dst = edge_index[1]
    ep = E_PAD - E
    src2d = jnp.concatenate(
        [src, jnp.zeros((ep,), jnp.int32)]).reshape(ECH, CH)
    dst2d = jnp.concatenate(
        [dst, N + (jnp.arange(ep, dtype=jnp.int32) % (N_PAD - N))]
    ).reshape(ECH, CH)
    batch2d = jnp.concatenate(
        [batch, G + (jnp.arange(N_PAD - N, dtype=jnp.int32) % (G_PAD - G))]
    ).reshape(BCH, CH)

    x_p = jnp.zeros((N_PAD, 8), _f32).at[:N, :IN_DIM].set(x)
    w1_p = jnp.zeros((8, HID), _f32).at[:IN_DIM].set(W1)

    zeros_col = jnp.zeros((N_PAD, DW), _f32)
    ones_col = jnp.ones((CH, DW), _f32)
    zeros_g = jnp.zeros((G_PAD, HH), _f32)

    deg = _deg_sc(dst2d, zeros_col, ones_col)
    dinv, ya, yb = _tc_first(deg, x_p, w1_p)

    for W, b in ((W2, b1), (W3, b2), (W4, b3), (W5, b4)):
        za, zb = _layer_sc(ya, yb, src2d, dst2d)
        ya, yb = _tc_layer(za, zb, dinv, b.reshape(1, HID), W)

    za, zb = _layer_sc(ya, yb, src2d, dst2d)
    ha, hb = _tc_hidden(za, zb, dinv, b5.reshape(1, HID))

    pooled = _pool_sc(ha, hb, batch2d, zeros_g)
    return _tc_out(pooled, Wl, bl.reshape(1, 2))
